# baseline (speedup 1.0000x reference)
import jax
import jax.numpy as jnp
from jax.experimental import pallas as pl

_F32 = jnp.float32


def _conv1(x, w72, bias, R):
    N, Cx, Hin, Win = x.shape
    Ho, Wo = Hin // 2, Win // 2
    Cout = w72.shape[-1]
    T = Ho // R
    grid = (N, T)
    rows_in = 2 * R + 2

    def body(tref, mref, boref, wref, bref, oref):
        i = pl.program_id(1)
        top = jnp.where(i > 0, tref[0, :, 7:8, :], jnp.zeros((Cx, 1, Win), _F32))
        bot = jnp.where(i < T - 1, boref[0, :, 0:1, :], jnp.zeros((Cx, 1, Win), _F32))
        xin = jnp.concatenate([top, mref[0], bot], axis=1)
        zc = jnp.zeros((Cx, rows_in, 1), _F32)
        xin = jnp.concatenate([zc, xin, zc], axis=2)
        xp8 = jnp.concatenate(
            [xin, jnp.zeros((8 - Cx, rows_in, Win + 2), _F32)], axis=0)
        t = jnp.transpose(xp8, (1, 2, 0))
        t = jnp.concatenate(
            [t, jnp.zeros((rows_in, Win + 2, 120), _F32)], axis=2)
        x2 = t.reshape(R + 1, 2, (Win + 2) // 2, 256)
        slabs = []
        for dy in range(3):
            ro, rp = dy // 2, dy % 2
            rows = x2[ro:ro + R, rp]
            for dx in range(3):
                co, cp = dx // 2, dx % 2
                slabs.append(jax.lax.slice(rows, (0, co, cp * 128),
                                           (R, co + Wo, cp * 128 + 8)))
        patches = jnp.concatenate(slabs, axis=2)
        acc = jnp.dot(patches.reshape(R * Wo, 72), wref[...],
                      preferred_element_type=_F32) + bref[0]
        oref[0] = jnp.maximum(acc, 0.0).reshape(R, Wo, Cout)

    return pl.pallas_call(
        body, grid=grid,
        in_specs=[
            pl.BlockSpec((1, Cx, 8, Win),
                         lambda n, i: (n, 0, jnp.maximum((2 * R * i - 1) // 8, 0), 0)),
            pl.BlockSpec((1, Cx, 2 * R, Win), lambda n, i: (n, 0, i, 0)),
            pl.BlockSpec((1, Cx, 8, Win),
                         lambda n, i: (n, 0, jnp.minimum((2 * R * i + 2 * R) // 8, Hin // 8 - 1), 0)),
            pl.BlockSpec(w72.shape, lambda n, i: (0, 0)),
            pl.BlockSpec((1, Cout), lambda n, i: (0, 0)),
        ],
        out_specs=pl.BlockSpec((1, R, Wo, Cout), lambda n, i: (n, i, 0, 0)),
        out_shape=jax.ShapeDtypeStruct((N, Ho, Wo, Cout), _F32),
    )(x, x, x, w72, bias.reshape(1, Cout))


def _haloed(tref, mref, boref, i, T, C):
    top = jnp.where(i > 0, tref[0], jnp.zeros_like(tref[0]))
    bot = jnp.where(i < T - 1, boref[0], jnp.zeros_like(boref[0]))
    xin = jnp.concatenate([top, mref[0], bot], axis=0)
    zc = jnp.zeros((xin.shape[0], 1, C), _F32)
    return jnp.concatenate([zc, xin, zc], axis=1)


def _s2conv(xh, wst, bias, R):
    N, Hin, Win, C = xh.shape
    Ho, Wo = Hin // 2, Win // 2
    Cout = wst.shape[-1]
    T = Ho // R
    grid = (N, T)

    def body(tref, mref, boref, wref, bref, oref):
        i = pl.program_id(1)
        xin = _haloed(tref, mref, boref, i, T, C)
        x2 = xin.reshape(R + 1, 2, (Win + 2) // 2, 2 * C)
        acc = jnp.zeros((R * Wo, Cout), _F32)
        for dy in range(3):
            ro, rp = dy // 2, dy % 2
            rows = x2[ro:ro + R, rp]
            for dx in range(3):
                co, cp = dx // 2, dx % 2
                sl = jax.lax.slice(rows, (0, co, cp * C),
                                   (R, co + Wo, (cp + 1) * C))
                acc = acc + jnp.dot(sl.reshape(R * Wo, C), wref[dy * 3 + dx],
                                    preferred_element_type=_F32)
        oref[0] = jnp.maximum(acc + bref[0], 0.0).reshape(R, Wo, Cout)

    return pl.pallas_call(
        body, grid=grid,
        in_specs=[
            pl.BlockSpec((1, 1, Win, C),
                         lambda n, i: (n, jnp.maximum(2 * R * i - 1, 0), 0, 0)),
            pl.BlockSpec((1, 2 * R, Win, C), lambda n, i: (n, i, 0, 0)),
            pl.BlockSpec((1, 1, Win, C),
                         lambda n, i: (n, jnp.minimum(2 * R * i + 2 * R, Hin - 1), 0, 0)),
            pl.BlockSpec(wst.shape, lambda n, i: (0, 0, 0)),
            pl.BlockSpec((1, Cout), lambda n, i: (0, 0)),
        ],
        out_specs=pl.BlockSpec((1, R, Wo, Cout), lambda n, i: (n, i, 0, 0)),
        out_shape=jax.ShapeDtypeStruct((N, Ho, Wo, Cout), _F32),
    )(xh, xh, xh, wst, bias.reshape(1, Cout))


def _upconv_weights(w):
    groups = {(0, 0): (0,), (0, 1): (1, 2), (1, 0): (0, 1), (1, 1): (2,)}
    mats = []
    for pi in range(2):
        for pj in range(2):
            for a in range(2):
                for b in range(2):
                    mats.append(sum(jnp.transpose(w[:, :, dy, dx])
                                    for dy in groups[(pi, a)]
                                    for dx in groups[(pj, b)]))
    return jnp.stack(mats)


def _upconv(g, w, bias, R):
    N, H, W, C = g.shape
    Cout = w.shape[0]
    wst = _upconv_weights(w)
    T = H // R
    grid = (N, T)

    def body(tref, mref, boref, wref, bref, oref):
        i = pl.program_id(1)
        gin = _haloed(tref, mref, boref, i, T, C)
        cols = [jax.lax.slice(gin, (0, d, 0), (R + 2, d + W, C))
                for d in range(3)]
        for pi in range(2):
            accs = []
            for pj in range(2):
                acc = jnp.zeros((R * W, Cout), _F32)
                for a in range(2):
                    for b in range(2):
                        sl = cols[pj + b][pi + a:pi + a + R]
                        widx = ((pi * 2 + pj) * 2 + a) * 2 + b
                        acc = acc + jnp.dot(sl.reshape(R * W, C), wref[widx],
                                            preferred_element_type=_F32)
                accs.append(jnp.maximum(acc + bref[0], 0.0).reshape(R, W, Cout))
            oref[0, :, pi] = jnp.concatenate(accs, axis=-1)

    return pl.pallas_call(
        body, grid=grid,
        in_specs=[
            pl.BlockSpec((1, 1, W, C),
                         lambda n, i: (n, jnp.maximum(R * i - 1, 0), 0, 0)),
            pl.BlockSpec((1, R, W, C), lambda n, i: (n, i, 0, 0)),
            pl.BlockSpec((1, 1, W, C),
                         lambda n, i: (n, jnp.minimum(R * i + R, H - 1), 0, 0)),
            pl.BlockSpec(wst.shape, lambda n, i: (0, 0, 0)),
            pl.BlockSpec((1, Cout), lambda n, i: (0, 0)),
        ],
        out_specs=pl.BlockSpec((1, R, 2, W, 2 * Cout),
                               lambda n, i: (n, i, 0, 0, 0)),
        out_shape=jax.ShapeDtypeStruct((N, H, 2, W, 2 * Cout), _F32),
    )(g, g, g, wst, bias.reshape(1, Cout)).reshape(N, 2 * H, 2 * W, Cout)


def _dec4(x3, wst, bias, xnchw, R):
    N, H, W, C = x3.shape
    Cout = wst.shape[-1]
    Co = xnchw.shape[1]
    T = H // R
    grid = (N, T)

    def body(tref, mref, boref, wref, bref, rref, oref, lref):
        i = pl.program_id(1)
        xin = _haloed(tref, mref, boref, i, T, C)
        cols = [jax.lax.slice(xin, (0, d, 0), (R + 2, d + W, C))
                for d in range(3)]
        acc = jnp.zeros((R * W, Cout), _F32)
        for dy in range(3):
            for dx in range(3):
                sl = cols[dx][dy:dy + R]
                acc = acc + jnp.dot(sl.reshape(R * W, C), wref[dy * 3 + dx],
                                    preferred_element_type=_F32)
        acc = acc + bref[0]
        out3 = jnp.transpose(acc).reshape(Cout, R, W)[:Co]
        oref[0] = out3
        dlt = out3 - rref[0]
        lref[0, 0] = jnp.full((8, 128), jnp.sum(dlt * dlt), _F32)

    return pl.pallas_call(
        body, grid=grid,
        in_specs=[
            pl.BlockSpec((1, 1, W, C),
                         lambda n, i: (n, jnp.maximum(R * i - 1, 0), 0, 0)),
            pl.BlockSpec((1, R, W, C), lambda n, i: (n, i, 0, 0)),
            pl.BlockSpec((1, 1, W, C),
                         lambda n, i: (n, jnp.minimum(R * i + R, H - 1), 0, 0)),
            pl.BlockSpec(wst.shape, lambda n, i: (0, 0, 0)),
            pl.BlockSpec((1, Cout), lambda n, i: (0, 0)),
            pl.BlockSpec((1, Co, R, W), lambda n, i: (n, 0, i, 0)),
        ],
        out_specs=[pl.BlockSpec((1, Co, R, W), lambda n, i: (n, 0, i, 0)),
                   pl.BlockSpec((1, 1, 8, 128), lambda n, i: (n, i, 0, 0))],
        out_shape=[jax.ShapeDtypeStruct((N, Co, H, W), _F32),
                   jax.ShapeDtypeStruct((N, T, 8, 128), _F32)],
    )(x3, x3, x3, wst, bias.reshape(1, Cout), xnchw)


def _vqcore(h2, w3m, b3, cb, w1m, b1, Mt):
    M, D = h2.shape
    K = cb.shape[0]
    grid = (M // Mt,)

    def body(href, w3r, b3r, cbr, w1r, b1r, oref):
        z = jnp.dot(href[...], w3r[...], preferred_element_type=_F32) + b3r[0]
        cbv = cbr[...]
        zz = jnp.sum(z * z, axis=1, keepdims=True)
        cc = jnp.sum(cbv * cbv, axis=1)
        cross = jax.lax.dot_general(z, cbv, (((1,), (1,)), ((), ())),
                                    preferred_element_type=_F32)
        d2 = zz - 2.0 * cross + cc[None, :]
        m = jnp.min(d2, axis=1, keepdims=True)
        ids = jax.lax.broadcasted_iota(jnp.int32, d2.shape, 1)
        idx = jnp.min(jnp.where(d2 == m, ids, K), axis=1, keepdims=True)
        q = jnp.dot((ids == idx).astype(_F32), cbv, preferred_element_type=_F32)
        q = z + (q - z)
        g = jnp.dot(q, w1r[...], preferred_element_type=_F32) + b1r[0]
        oref[...] = jnp.maximum(g, 0.0)

    return pl.pallas_call(
        body, grid=grid,
        in_specs=[pl.BlockSpec((Mt, D), lambda i: (i, 0)),
                  pl.BlockSpec(w3m.shape, lambda i: (0, 0)),
                  pl.BlockSpec((1, w3m.shape[1]), lambda i: (0, 0)),
                  pl.BlockSpec(cb.shape, lambda i: (0, 0)),
                  pl.BlockSpec(w1m.shape, lambda i: (0, 0)),
                  pl.BlockSpec((1, w1m.shape[1]), lambda i: (0, 0))],
        out_specs=pl.BlockSpec((Mt, w1m.shape[1]), lambda i: (i, 0)),
        out_shape=jax.ShapeDtypeStruct((M, w1m.shape[1]), _F32),
    )(h2, w3m, b3.reshape(1, -1), cb, w1m, b1.reshape(1, -1))


def _tapw(w, dy, dx):
    return jnp.transpose(w[:, :, dy, dx])


def kernel(x, enc_w1, enc_b1, enc_w2, enc_b2, enc_w3, enc_b3, codebook,
           dec_w1, dec_b1, dec_w2, dec_b2, dec_w3, dec_b3, dec_w4, dec_b4):
    N = x.shape[0]

    w72 = jnp.pad(jnp.transpose(enc_w1, (2, 3, 1, 0)),
                  ((0, 0), (0, 0), (0, 5), (0, 0))).reshape(72, -1)
    h1 = _conv1(x, w72, enc_b1, R=28)

    ws2 = jnp.stack([_tapw(enc_w2, dy, dx) for dy in range(3) for dx in range(3)])
    h2 = _s2conv(h1, ws2, enc_b2, R=28)

    g = _vqcore(h2.reshape(-1, 128), jnp.transpose(enc_w3[:, :, 0, 0]), enc_b3,
                codebook, jnp.transpose(dec_w1[:, :, 0, 0]), dec_b1, Mt=3136)
    g = g.reshape(N, 56, 56, -1)

    g2 = _upconv(g, dec_w2, dec_b2, R=28)
    g3 = _upconv(g2, dec_w3, dec_b3, R=28)

    w4p = jnp.pad(dec_w4, ((0, 5), (0, 0), (0, 0), (0, 0)))
    ws4 = jnp.stack([_tapw(w4p, dy, dx) for dy in range(3) for dx in range(3)])
    out, parts = _dec4(g3, ws4, jnp.pad(dec_b4, (0, 5)), x, R=32)

    quantized = out
    loss = jnp.sum(parts) / (8.0 * 128.0) / jnp.float32(x.size)
    return (quantized, loss, jnp.float32(0.25) * loss)


# speedup vs baseline: 1.0448x; 1.0448x over previous
import jax
import jax.numpy as jnp
from jax.experimental import pallas as pl

_F32 = jnp.float32


def _conv1(x, w72, bias, R):
    N, Cx, Hin, Win = x.shape
    Ho, Wo = Hin // 2, Win // 2
    Cout = w72.shape[-1]
    T = Ho // R
    grid = (N, T)
    rows_in = 2 * R + 2

    def body(tref, mref, boref, wref, bref, oref):
        i = pl.program_id(1)
        top = jnp.where(i > 0, tref[0, :, 7:8, :], jnp.zeros((Cx, 1, Win), _F32))
        bot = jnp.where(i < T - 1, boref[0, :, 0:1, :], jnp.zeros((Cx, 1, Win), _F32))
        xin = jnp.concatenate([top, mref[0], bot], axis=1)
        zc = jnp.zeros((Cx, rows_in, 1), _F32)
        xin = jnp.concatenate([zc, xin, zc], axis=2)
        xp8 = jnp.concatenate(
            [xin, jnp.zeros((8 - Cx, rows_in, Win + 2), _F32)], axis=0)
        t = jnp.transpose(xp8, (1, 2, 0))
        t = jnp.concatenate(
            [t, jnp.zeros((rows_in, Win + 2, 120), _F32)], axis=2)
        x2 = t.reshape(R + 1, 2, (Win + 2) // 2, 256)
        slabs = []
        for dy in range(3):
            ro, rp = dy // 2, dy % 2
            rows = x2[ro:ro + R, rp]
            for dx in range(3):
                co, cp = dx // 2, dx % 2
                slabs.append(jax.lax.slice(rows, (0, co, cp * 128),
                                           (R, co + Wo, cp * 128 + 8)))
        patches = jnp.concatenate(slabs, axis=2)
        acc = jnp.dot(patches.reshape(R * Wo, 72), wref[...],
                      preferred_element_type=_F32) + bref[0]
        oref[0] = jnp.maximum(acc, 0.0).reshape(R, Wo, Cout)

    return pl.pallas_call(
        body, grid=grid,
        in_specs=[
            pl.BlockSpec((1, Cx, 8, Win),
                         lambda n, i: (n, 0, jnp.maximum((2 * R * i - 1) // 8, 0), 0)),
            pl.BlockSpec((1, Cx, 2 * R, Win), lambda n, i: (n, 0, i, 0)),
            pl.BlockSpec((1, Cx, 8, Win),
                         lambda n, i: (n, 0, jnp.minimum((2 * R * i + 2 * R) // 8, Hin // 8 - 1), 0)),
            pl.BlockSpec(w72.shape, lambda n, i: (0, 0)),
            pl.BlockSpec((1, Cout), lambda n, i: (0, 0)),
        ],
        out_specs=pl.BlockSpec((1, R, Wo, Cout), lambda n, i: (n, i, 0, 0)),
        out_shape=jax.ShapeDtypeStruct((N, Ho, Wo, Cout), _F32),
    )(x, x, x, w72, bias.reshape(1, Cout))


def _haloed(tref, mref, boref, i, T, C):
    top = jnp.where(i > 0, tref[0], jnp.zeros_like(tref[0]))
    bot = jnp.where(i < T - 1, boref[0], jnp.zeros_like(boref[0]))
    xin = jnp.concatenate([top, mref[0], bot], axis=0)
    zc = jnp.zeros((xin.shape[0], 1, C), xin.dtype)
    return jnp.concatenate([zc, xin, zc], axis=1)


def _s2conv(xh, wst, bias, R):
    N, Hin, Win, C = xh.shape
    Ho, Wo = Hin // 2, Win // 2
    Cout = wst.shape[-1]
    T = Ho // R
    grid = (N, T)

    def body(tref, mref, boref, wref, bref, oref):
        i = pl.program_id(1)
        xin = _haloed(tref, mref, boref, i, T, C)
        x2 = xin.reshape(R + 1, 2, (Win + 2) // 2, 2 * C)
        acc = jnp.zeros((R * Wo, Cout), _F32)
        for dy in range(3):
            ro, rp = dy // 2, dy % 2
            rows = x2[ro:ro + R, rp]
            for dx in range(3):
                co, cp = dx // 2, dx % 2
                sl = jax.lax.slice(rows, (0, co, cp * C),
                                   (R, co + Wo, (cp + 1) * C))
                acc = acc + jnp.dot(sl.reshape(R * Wo, C), wref[dy * 3 + dx],
                                    preferred_element_type=_F32)
        oref[0] = jnp.maximum(acc + bref[0], 0.0).reshape(R, Wo, Cout)

    return pl.pallas_call(
        body, grid=grid,
        in_specs=[
            pl.BlockSpec((1, 1, Win, C),
                         lambda n, i: (n, jnp.maximum(2 * R * i - 1, 0), 0, 0)),
            pl.BlockSpec((1, 2 * R, Win, C), lambda n, i: (n, i, 0, 0)),
            pl.BlockSpec((1, 1, Win, C),
                         lambda n, i: (n, jnp.minimum(2 * R * i + 2 * R, Hin - 1), 0, 0)),
            pl.BlockSpec(wst.shape, lambda n, i: (0, 0, 0)),
            pl.BlockSpec((1, Cout), lambda n, i: (0, 0)),
        ],
        out_specs=pl.BlockSpec((1, R, Wo, Cout), lambda n, i: (n, i, 0, 0)),
        out_shape=jax.ShapeDtypeStruct((N, Ho, Wo, Cout), _F32),
    )(xh, xh, xh, wst, bias.reshape(1, Cout))


def _upconv_weights(w):
    groups = {(0, 0): (0,), (0, 1): (1, 2), (1, 0): (0, 1), (1, 1): (2,)}
    mats = []
    for pi in range(2):
        for pj in range(2):
            for a in range(2):
                for b in range(2):
                    mats.append(sum(jnp.transpose(w[:, :, dy, dx])
                                    for dy in groups[(pi, a)]
                                    for dx in groups[(pj, b)]))
    return jnp.stack(mats)


def _upconv(g, w, bias, R):
    N, H, W, C = g.shape
    Cout = w.shape[0]
    wst = _upconv_weights(w)
    T = H // R
    grid = (N, T)

    def body(tref, mref, boref, wref, bref, oref):
        i = pl.program_id(1)
        gin = _haloed(tref, mref, boref, i, T, C).astype(_F32)
        cols = [jax.lax.slice(gin, (0, d, 0), (R + 2, d + W, C))
                for d in range(3)]
        for pi in range(2):
            accs = []
            for pj in range(2):
                acc = jnp.zeros((R * W, Cout), _F32)
                for a in range(2):
                    for b in range(2):
                        sl = cols[pj + b][pi + a:pi + a + R]
                        widx = ((pi * 2 + pj) * 2 + a) * 2 + b
                        acc = acc + jnp.dot(sl.reshape(R * W, C), wref[widx],
                                            preferred_element_type=_F32)
                accs.append(jnp.maximum(acc + bref[0], 0.0).reshape(R, W, Cout))
            oref[0, :, pi] = jnp.concatenate(accs, axis=-1).astype(jnp.bfloat16)

    return pl.pallas_call(
        body, grid=grid,
        in_specs=[
            pl.BlockSpec((1, 1, W, C),
                         lambda n, i: (n, jnp.maximum(R * i - 1, 0), 0, 0)),
            pl.BlockSpec((1, R, W, C), lambda n, i: (n, i, 0, 0)),
            pl.BlockSpec((1, 1, W, C),
                         lambda n, i: (n, jnp.minimum(R * i + R, H - 1), 0, 0)),
            pl.BlockSpec(wst.shape, lambda n, i: (0, 0, 0)),
            pl.BlockSpec((1, Cout), lambda n, i: (0, 0)),
        ],
        out_specs=pl.BlockSpec((1, R, 2, W, 2 * Cout),
                               lambda n, i: (n, i, 0, 0, 0)),
        out_shape=jax.ShapeDtypeStruct((N, H, 2, W, 2 * Cout), jnp.bfloat16),
    )(g, g, g, wst, bias.reshape(1, Cout)).reshape(N, 2 * H, 2 * W, Cout)


def _dec4(x3, wst, bias, xnchw, R):
    N, H, W, C = x3.shape
    Cout = wst.shape[-1]
    Co = xnchw.shape[1]
    T = H // R
    grid = (N, T)

    def body(tref, mref, boref, wref, bref, rref, oref, lref):
        i = pl.program_id(1)
        xin = _haloed(tref, mref, boref, i, T, C).astype(_F32)
        cols = [jax.lax.slice(xin, (0, d, 0), (R + 2, d + W, C))
                for d in range(3)]
        acc = jnp.zeros((R * W, Cout), _F32)
        for dy in range(3):
            for dx in range(3):
                sl = cols[dx][dy:dy + R]
                acc = acc + jnp.dot(sl.reshape(R * W, C), wref[dy * 3 + dx],
                                    preferred_element_type=_F32)
        acc = acc + bref[0]
        out3 = jnp.transpose(acc).reshape(Cout, R, W)[:Co]
        oref[0] = out3
        dlt = out3 - rref[0]
        lref[0, 0] = jnp.full((8, 128), jnp.sum(dlt * dlt), _F32)

    return pl.pallas_call(
        body, grid=grid,
        in_specs=[
            pl.BlockSpec((1, 1, W, C),
                         lambda n, i: (n, jnp.maximum(R * i - 1, 0), 0, 0)),
            pl.BlockSpec((1, R, W, C), lambda n, i: (n, i, 0, 0)),
            pl.BlockSpec((1, 1, W, C),
                         lambda n, i: (n, jnp.minimum(R * i + R, H - 1), 0, 0)),
            pl.BlockSpec(wst.shape, lambda n, i: (0, 0, 0)),
            pl.BlockSpec((1, Cout), lambda n, i: (0, 0)),
            pl.BlockSpec((1, Co, R, W), lambda n, i: (n, 0, i, 0)),
        ],
        out_specs=[pl.BlockSpec((1, Co, R, W), lambda n, i: (n, 0, i, 0)),
                   pl.BlockSpec((1, 1, 8, 128), lambda n, i: (n, i, 0, 0))],
        out_shape=[jax.ShapeDtypeStruct((N, Co, H, W), _F32),
                   jax.ShapeDtypeStruct((N, T, 8, 128), _F32)],
    )(x3, x3, x3, wst, bias.reshape(1, Cout), xnchw)


def _vqcore(h2, w3m, b3, cb, w1m, b1, Mt):
    M, D = h2.shape
    K = cb.shape[0]
    grid = (M // Mt,)

    def body(href, w3r, b3r, cbr, w1r, b1r, oref):
        z = jnp.dot(href[...], w3r[...], preferred_element_type=_F32) + b3r[0]
        cbv = cbr[...]
        zz = jnp.sum(z * z, axis=1, keepdims=True)
        cc = jnp.sum(cbv * cbv, axis=1)
        cross = jax.lax.dot_general(z, cbv, (((1,), (1,)), ((), ())),
                                    preferred_element_type=_F32)
        d2 = zz - 2.0 * cross + cc[None, :]
        m = jnp.min(d2, axis=1, keepdims=True)
        ids = jax.lax.broadcasted_iota(jnp.int32, d2.shape, 1)
        idx = jnp.min(jnp.where(d2 == m, ids, K), axis=1, keepdims=True)
        q = jnp.dot((ids == idx).astype(_F32), cbv, preferred_element_type=_F32)
        q = z + (q - z)
        g = jnp.dot(q, w1r[...], preferred_element_type=_F32) + b1r[0]
        oref[...] = jnp.maximum(g, 0.0)

    return pl.pallas_call(
        body, grid=grid,
        in_specs=[pl.BlockSpec((Mt, D), lambda i: (i, 0)),
                  pl.BlockSpec(w3m.shape, lambda i: (0, 0)),
                  pl.BlockSpec((1, w3m.shape[1]), lambda i: (0, 0)),
                  pl.BlockSpec(cb.shape, lambda i: (0, 0)),
                  pl.BlockSpec(w1m.shape, lambda i: (0, 0)),
                  pl.BlockSpec((1, w1m.shape[1]), lambda i: (0, 0))],
        out_specs=pl.BlockSpec((Mt, w1m.shape[1]), lambda i: (i, 0)),
        out_shape=jax.ShapeDtypeStruct((M, w1m.shape[1]), _F32),
    )(h2, w3m, b3.reshape(1, -1), cb, w1m, b1.reshape(1, -1))


def _tapw(w, dy, dx):
    return jnp.transpose(w[:, :, dy, dx])


def kernel(x, enc_w1, enc_b1, enc_w2, enc_b2, enc_w3, enc_b3, codebook,
           dec_w1, dec_b1, dec_w2, dec_b2, dec_w3, dec_b3, dec_w4, dec_b4):
    N = x.shape[0]

    w72 = jnp.pad(jnp.transpose(enc_w1, (2, 3, 1, 0)),
                  ((0, 0), (0, 0), (0, 5), (0, 0))).reshape(72, -1)
    h1 = _conv1(x, w72, enc_b1, R=28)

    ws2 = jnp.stack([_tapw(enc_w2, dy, dx) for dy in range(3) for dx in range(3)])
    h2 = _s2conv(h1, ws2, enc_b2, R=28)

    g = _vqcore(h2.reshape(-1, 128), jnp.transpose(enc_w3[:, :, 0, 0]), enc_b3,
                codebook, jnp.transpose(dec_w1[:, :, 0, 0]), dec_b1, Mt=3136)
    g = g.reshape(N, 56, 56, -1)

    g2 = _upconv(g, dec_w2, dec_b2, R=28)
    g3 = _upconv(g2, dec_w3, dec_b3, R=28)

    w4p = jnp.pad(dec_w4, ((0, 5), (0, 0), (0, 0), (0, 0)))
    ws4 = jnp.stack([_tapw(w4p, dy, dx) for dy in range(3) for dx in range(3)])
    out, parts = _dec4(g3, ws4, jnp.pad(dec_b4, (0, 5)), x, R=32)

    quantized = out
    loss = jnp.sum(parts) / (8.0 * 128.0) / jnp.float32(x.size)
    return (quantized, loss, jnp.float32(0.25) * loss)
